# baseline (speedup 1.0000x reference)
import functools

import jax
import jax.numpy as jnp
from jax import lax
from jax.experimental import pallas as pl
from jax.experimental.pallas import tpu as pltpu
from jax.experimental.pallas import tpu_sc as plsc

N_NODES = 10000
FDIM = 128
NCORES = 2
NSUB = 16
NTILES = NCORES * NSUB
CHUNK = 128
NCHUNKS = 80
HALF = NCHUNKS // 2
ACC_ROWS = N_NODES + 8
ROWS_PER_TILE = 624
TAIL_START = NSUB * ROWS_PER_TILE
TAIL_ROWS = N_NODES - TAIL_START


def _quat_mul_mat(k):
    r, i, j, q = jnp.split(k, 4, axis=1)
    r2 = jnp.concatenate([r, -i, -j, -q], axis=0)
    i2 = jnp.concatenate([i, r, -q, j], axis=0)
    j2 = jnp.concatenate([j, q, r, -i], axis=0)
    k2 = jnp.concatenate([q, -j, i, r], axis=0)
    return jnp.concatenate([r2, i2, j2, k2], axis=1)


def _sc_aggregate(x, src, dst):
    mesh = plsc.VectorSubcoreMesh(core_axis_name="c", subcore_axis_name="s")

    @functools.partial(
        pl.kernel,
        out_type=jax.ShapeDtypeStruct((NCORES, N_NODES, FDIM), jnp.float32),
        mesh=mesh,
        scratch_types=[
            pltpu.VMEM((HALF, CHUNK), jnp.int32),
            pltpu.VMEM((HALF, CHUNK), jnp.int32),
            pltpu.VMEM((CHUNK, FDIM), jnp.float32),
            pltpu.VMEM((CHUNK, FDIM), jnp.float32),
            pltpu.VMEM_SHARED((ACC_ROWS, FDIM), jnp.float32),
            pltpu.SemaphoreType.DMA,
            pltpu.SemaphoreType.DMA,
        ],
    )
    def body(x_hbm, src_hbm, dst_hbm, out_hbm, src_v, dst_v, rows0, rows1,
             acc, sem0, sem1):
        c = lax.axis_index("c")
        s = lax.axis_index("s")
        wid = c * NSUB + s
        bufs = (rows0, rows1)
        sems = (sem0, sem1)

        zero16 = jnp.zeros((16,), jnp.float32)

        def zero_row(i, carry):
            for j in range(FDIM // 16):
                rows0[i, pl.ds(j * 16, 16)] = zero16
            return carry

        lax.fori_loop(0, CHUNK, zero_row, 0)

        row0 = s * ROWS_PER_TILE
        nz_full = ROWS_PER_TILE // CHUNK
        nz_rem = ROWS_PER_TILE % CHUNK

        def zero_acc(i, carry):
            pltpu.sync_copy(rows0, acc.at[pl.ds(row0 + i * CHUNK, CHUNK)])
            return carry

        lax.fori_loop(0, nz_full, zero_acc, 0)
        if nz_rem:
            pltpu.sync_copy(
                rows0.at[pl.ds(0, nz_rem)],
                acc.at[pl.ds(row0 + nz_full * CHUNK, nz_rem)],
            )

        @pl.when(s == NSUB - 1)
        def _():
            pltpu.sync_copy(
                rows0.at[pl.ds(0, TAIL_ROWS)],
                acc.at[pl.ds(TAIL_START, TAIL_ROWS)],
            )

        plsc.subcore_barrier()

        def gstart(chunk_idx, b):
            pltpu.async_copy(x_hbm.at[src_v.at[chunk_idx]], bufs[b], sems[b])

        def gwait(chunk_idx, b):
            pltpu.make_async_copy(
                x_hbm.at[src_v.at[chunk_idx]], bufs[b], sems[b]
            ).wait()

        for h in range(NCHUNKS // HALF):
            pltpu.sync_copy(src_hbm.at[wid, pl.ds(h * HALF, HALF)], src_v)
            pltpu.sync_copy(dst_hbm.at[wid, pl.ds(h * HALF, HALF)], dst_v)

            def step(i, carry):
                pltpu.async_copy(
                    x_hbm.at[src_v.at[i]], rows0, sem0
                ).wait()
                pltpu.sync_copy(rows0, acc.at[dst_v.at[i]], add=True)
                return carry

            lax.fori_loop(0, HALF, step, 0)
        plsc.subcore_barrier()

        pltpu.sync_copy(
            acc.at[pl.ds(row0, ROWS_PER_TILE)],
            out_hbm.at[c, pl.ds(row0, ROWS_PER_TILE)],
        )

        @pl.when(s == NSUB - 1)
        def _():
            pltpu.sync_copy(
                acc.at[pl.ds(TAIL_START, TAIL_ROWS)],
                out_hbm.at[c, pl.ds(TAIL_START, TAIL_ROWS)],
            )

    return body(x, src, dst)


def _tc_finish(partials, A, B, gamma2d, beta2d):
    def body(p_ref, a_ref, b_ref, g_ref, bt_ref, o_ref):
        agg = p_ref[0] + p_ref[1]
        a_h = _quat_mul_mat(a_ref[...])
        b_h = _quat_mul_mat(b_ref[...])
        zeros = jnp.zeros_like(a_h)
        w = jnp.concatenate(
            [
                jnp.concatenate([a_h, b_h], axis=1),
                jnp.concatenate([zeros, a_h], axis=1),
            ],
            axis=0,
        )
        s = lax.dot_general(
            agg,
            w,
            (((1,), (0,)), ((), ())),
            preferred_element_type=jnp.float32,
            precision=lax.Precision.HIGHEST,
        )
        mean = jnp.mean(s, axis=0, keepdims=True)
        d = s - mean
        var = jnp.mean(d * d, axis=0, keepdims=True)
        o_ref[...] = jnp.tanh(
            d * lax.rsqrt(var + 1e-5) * g_ref[...] + bt_ref[...]
        )

    return pl.pallas_call(
        body,
        out_shape=jax.ShapeDtypeStruct((N_NODES, FDIM), jnp.float32),
    )(partials, A, B, gamma2d, beta2d)


def kernel(input, edge_index, A, B, gamma, beta):
    ei = edge_index.astype(jnp.int32)
    n_pad = NTILES * NCHUNKS * CHUNK - ei.shape[1]
    dst = jnp.concatenate(
        [ei[0], jnp.full((n_pad,), N_NODES, jnp.int32)]
    ).reshape(NTILES, NCHUNKS, CHUNK)
    src = jnp.concatenate(
        [ei[1], jnp.zeros((n_pad,), jnp.int32)]
    ).reshape(NTILES, NCHUNKS, CHUNK)
    partials = _sc_aggregate(input, src, dst)
    return _tc_finish(
        partials, A, B, gamma.reshape(1, FDIM), beta.reshape(1, FDIM)
    )


# speedup vs baseline: 1.4605x; 1.4605x over previous
import functools

import jax
import jax.numpy as jnp
from jax import lax
from jax.experimental import pallas as pl
from jax.experimental.pallas import tpu as pltpu
from jax.experimental.pallas import tpu_sc as plsc

N_NODES = 10000
FDIM = 128
NCORES = 2
NSUB = 16
NTILES = NCORES * NSUB
CHUNK = 128
NCHUNKS = 80
HALF = NCHUNKS // 2
ACC_ROWS = N_NODES + 8
ROWS_PER_TILE = 624
TAIL_START = NSUB * ROWS_PER_TILE
TAIL_ROWS = N_NODES - TAIL_START


def _quat_mul_mat(k):
    r, i, j, q = jnp.split(k, 4, axis=1)
    r2 = jnp.concatenate([r, -i, -j, -q], axis=0)
    i2 = jnp.concatenate([i, r, -q, j], axis=0)
    j2 = jnp.concatenate([j, q, r, -i], axis=0)
    k2 = jnp.concatenate([q, -j, i, r], axis=0)
    return jnp.concatenate([r2, i2, j2, k2], axis=1)


def _sc_aggregate(x, src, dst):
    mesh = plsc.VectorSubcoreMesh(core_axis_name="c", subcore_axis_name="s")

    @functools.partial(
        pl.kernel,
        out_type=jax.ShapeDtypeStruct((NCORES, N_NODES, FDIM), jnp.float32),
        mesh=mesh,
        scratch_types=[
            pltpu.VMEM((HALF, CHUNK), jnp.int32),
            pltpu.VMEM((HALF, CHUNK), jnp.int32),
            pltpu.VMEM((CHUNK, FDIM), jnp.float32),
            pltpu.VMEM((CHUNK, FDIM), jnp.float32),
            pltpu.VMEM_SHARED((ACC_ROWS, FDIM), jnp.float32),
            pltpu.SemaphoreType.DMA,
            pltpu.SemaphoreType.DMA,
        ],
    )
    def body(x_hbm, src_hbm, dst_hbm, out_hbm, src_v, dst_v, rows0, rows1,
             acc, sem0, sem1):
        c = lax.axis_index("c")
        s = lax.axis_index("s")
        wid = c * NSUB + s
        bufs = (rows0, rows1)
        sems = (sem0, sem1)

        zero16 = jnp.zeros((16,), jnp.float32)

        def zero_row(i, carry):
            for j in range(FDIM // 16):
                rows0[i, pl.ds(j * 16, 16)] = zero16
            return carry

        lax.fori_loop(0, CHUNK, zero_row, 0)

        row0 = s * ROWS_PER_TILE
        nz_full = ROWS_PER_TILE // CHUNK
        nz_rem = ROWS_PER_TILE % CHUNK

        def zero_acc(i, carry):
            pltpu.sync_copy(rows0, acc.at[pl.ds(row0 + i * CHUNK, CHUNK)])
            return carry

        lax.fori_loop(0, nz_full, zero_acc, 0)
        if nz_rem:
            pltpu.sync_copy(
                rows0.at[pl.ds(0, nz_rem)],
                acc.at[pl.ds(row0 + nz_full * CHUNK, nz_rem)],
            )

        @pl.when(s == NSUB - 1)
        def _():
            pltpu.sync_copy(
                rows0.at[pl.ds(0, TAIL_ROWS)],
                acc.at[pl.ds(TAIL_START, TAIL_ROWS)],
            )

        plsc.subcore_barrier()

        def gstart(chunk_idx, b):
            pltpu.async_copy(x_hbm.at[src_v.at[chunk_idx]], bufs[b], sems[b])

        def gwait(chunk_idx, b):
            pltpu.make_async_copy(
                x_hbm.at[src_v.at[chunk_idx]], bufs[b], sems[b]
            ).wait()

        n_iter = HALF // 2
        for h in range(NCHUNKS // HALF):
            pltpu.sync_copy(src_hbm.at[wid, pl.ds(h * HALF, HALF)], src_v)
            pltpu.sync_copy(dst_hbm.at[wid, pl.ds(h * HALF, HALF)], dst_v)
            gstart(0, 0)
            gstart(1, 1)

            def step(it, carry):
                i0 = it * 2
                for b in range(2):
                    i = i0 + b
                    gwait(i, b)
                    pltpu.sync_copy(bufs[b], acc.at[dst_v.at[i]], add=True)

                    @pl.when(it < n_iter - 1)
                    def _():
                        gstart(i + 2, b)

                return carry

            lax.fori_loop(0, n_iter, step, 0)
        plsc.subcore_barrier()

        pltpu.sync_copy(
            acc.at[pl.ds(row0, ROWS_PER_TILE)],
            out_hbm.at[c, pl.ds(row0, ROWS_PER_TILE)],
        )

        @pl.when(s == NSUB - 1)
        def _():
            pltpu.sync_copy(
                acc.at[pl.ds(TAIL_START, TAIL_ROWS)],
                out_hbm.at[c, pl.ds(TAIL_START, TAIL_ROWS)],
            )

    return body(x, src, dst)


def _tc_finish(partials, A, B, gamma2d, beta2d):
    def body(p_ref, a_ref, b_ref, g_ref, bt_ref, o_ref):
        agg = p_ref[0] + p_ref[1]
        a_h = _quat_mul_mat(a_ref[...])
        b_h = _quat_mul_mat(b_ref[...])
        zeros = jnp.zeros_like(a_h)
        w = jnp.concatenate(
            [
                jnp.concatenate([a_h, b_h], axis=1),
                jnp.concatenate([zeros, a_h], axis=1),
            ],
            axis=0,
        )
        s = lax.dot_general(
            agg,
            w,
            (((1,), (0,)), ((), ())),
            preferred_element_type=jnp.float32,
            precision=lax.Precision.HIGHEST,
        )
        mean = jnp.mean(s, axis=0, keepdims=True)
        d = s - mean
        var = jnp.mean(d * d, axis=0, keepdims=True)
        o_ref[...] = jnp.tanh(
            d * lax.rsqrt(var + 1e-5) * g_ref[...] + bt_ref[...]
        )

    return pl.pallas_call(
        body,
        out_shape=jax.ShapeDtypeStruct((N_NODES, FDIM), jnp.float32),
    )(partials, A, B, gamma2d, beta2d)


def kernel(input, edge_index, A, B, gamma, beta):
    ei = edge_index.astype(jnp.int32)
    n_edges = ei.shape[1]
    per_tile = n_edges // NTILES
    pad_per_tile = NCHUNKS * CHUNK - per_tile
    pad_dst = jnp.broadcast_to(
        N_NODES + (jnp.arange(pad_per_tile, dtype=jnp.int32) % 8),
        (NTILES, pad_per_tile),
    )
    pad_src = jnp.zeros((NTILES, pad_per_tile), jnp.int32)
    dst = jnp.concatenate(
        [ei[0].reshape(NTILES, per_tile), pad_dst], axis=1
    ).reshape(NTILES, NCHUNKS, CHUNK)
    src = jnp.concatenate(
        [ei[1].reshape(NTILES, per_tile), pad_src], axis=1
    ).reshape(NTILES, NCHUNKS, CHUNK)
    partials = _sc_aggregate(input, src, dst)
    return _tc_finish(
        partials, A, B, gamma.reshape(1, FDIM), beta.reshape(1, FDIM)
    )
